# baseline (speedup 1.0000x reference)
import functools
import jax
import jax.numpy as jnp
from jax import lax
from jax.experimental import pallas as pl
from jax.experimental.pallas import tpu as pltpu
from jax.experimental.pallas import tpu_sc as plsc

K = 3
NLAT = 128
NLON = 256
NNZ = 32
CIN = 64
COUT = 64
NSEG = K * NLAT
NWORKERS = 32
NROWS = NLON // 4
ROWLEN = 2 * CIN
LANES = 16
LAT_BLK = 8
B_TC = 8
SEG_BASE = K * LAT_BLK * B_TC
SC_SEGS = NSEG - SEG_BASE


def _disco_tc_kernel(idx_ref, val_ref, xt_ref, w2_ref, bias_ref, out_ref,
                     y_ref):
  UNROLL = 8
  for k in range(K):
    for j in range(LAT_BLK):
      def body(n2, accs):
        a0, a1 = accs
        for u in range(UNROLL):
          n = n2 * UNROLL + u
          iflat = idx_ref[k, j, n]
          lat = iflat >> 8
          shift = (NLON - (iflat & (NLON - 1))) & (NLON - 1)
          val = val_ref[k, j, n]
          t = val * pltpu.roll(xt_ref[lat], shift, 1)
          if u % 2 == 0:
            a0 = a0 + t
          else:
            a1 = a1 + t
        return (a0, a1)

      z = jnp.zeros((CIN, NLON), jnp.float32)
      a0, a1 = lax.fori_loop(0, NNZ // UNROLL, body, (z, z))
      y_ref[k * CIN:(k + 1) * CIN, j * NLON:(j + 1) * NLON] = a0 + a1
  out_ref[...] = (
      jnp.dot(w2_ref[...], y_ref[...], preferred_element_type=jnp.float32)
      + bias_ref[...]
  )


def _sc_stage_a(xp, iflat_x, vals_x, y_out,
                if_v, val_v, idx_b, gbuf, acc, sem0, sem1):
  wid = lax.axis_index("s") * 2 + lax.axis_index("c")
  iota = lax.iota(jnp.int32, LANES)
  sems = (sem0, sem1)
  n_full = SC_SEGS // NWORKERS
  rem = SC_SEGS % NWORKERS
  n_t = n_full + jnp.where(wid < rem, 1, 0)

  def build_and_fire(e, slot):
    isp = if_v[e]
    lat_hi = (isp >> 8) << 8
    s_s = isp & (NLON - 1)
    for c in range(NROWS // LANES):
      idx_b[slot, pl.ds(c * LANES, LANES)] = (
          lat_hi | ((s_s + 4 * (c * LANES + iota)) & (NLON - 1)))
    pltpu.make_async_copy(
        xp.at[idx_b.at[slot]], gbuf.at[slot], sems[slot]).start()

  def wait(slot):
    pltpu.make_async_copy(
        xp.at[idx_b.at[slot]], gbuf.at[slot], sems[slot]).wait()

  def fma(e, slot):
    vsp = val_v[e]

    hi_mask = jnp.full((LANES,), -65536, jnp.int32)
    sh16 = jnp.full((LANES,), 16, jnp.int32)

    @plsc.parallel_loop(0, NROWS, unroll=4)
    def chunk_body(i):
      for q in range(4):
        for c in range(2):
          g = gbuf[slot, i, pl.ds(q * 2 * LANES + c * LANES, LANES)]
          a = lax.bitcast_convert_type(lax.shift_left(g, sh16), jnp.float32)
          b = lax.bitcast_convert_type(
              lax.bitwise_and(g, hi_mask), jnp.float32)
          plsc.addupdate(
              acc.at[4 * i + q, pl.ds(c * 2 * LANES, LANES)], a * vsp)
          plsc.addupdate(
              acc.at[4 * i + q, pl.ds(c * 2 * LANES + LANES, LANES)],
              b * vsp)

  def seg_body(t, _):
    seg = t * NWORKERS + wid
    pltpu.sync_copy(iflat_x.at[seg], if_v)
    pltpu.sync_copy(vals_x.at[seg], val_v)

    zeros = jnp.zeros((LANES,), jnp.float32)

    @plsc.parallel_loop(0, NLON, unroll=8)
    def zero_body(r):
      for c in range(CIN // LANES):
        acc[r, pl.ds(c * LANES, LANES)] = zeros

    build_and_fire(0, 0)
    def pair_body(m, _):
      e0 = 2 * m
      e1 = e0 + 1
      build_and_fire(e1, 1)
      wait(0)
      fma(e0, 0)

      @pl.when(e1 < NNZ - 1)
      def _():
        build_and_fire(e1 + 1, 0)
      wait(1)
      fma(e1, 1)
      return 0
    lax.fori_loop(0, NNZ // 2, pair_body, 0)

    pltpu.sync_copy(acc, y_out.at[seg])
    return 0

  lax.fori_loop(0, n_t, seg_body, 0)


def _mix_kernel(y_ref, wkt_ref, bias_ref, out_ref):
  for j in range(LAT_BLK):
    acc = None
    for k in range(K):
      t2 = y_ref[j * K + k]
      p = jnp.dot(t2, wkt_ref[k], preferred_element_type=jnp.float32)
      acc = p if acc is None else acc + p
    out_ref[:, j * NLON:(j + 1) * NLON] = acc.T + bias_ref[...]


@jax.jit
def kernel(x, psi_idx, psi_vals, weight, bias):
  xt = jnp.transpose(x[0], (1, 0, 2))
  xlc = jnp.transpose(x[0], (1, 2, 0))
  xquad = jnp.concatenate(
      [xlc, jnp.roll(xlc, -1, axis=1), jnp.roll(xlc, -2, axis=1),
       jnp.roll(xlc, -3, axis=1)], axis=-1)
  xq16 = xquad.astype(jnp.bfloat16).reshape(NLAT * NLON, ROWLEN, 2)
  xp = lax.bitcast_convert_type(xq16, jnp.int32)

  idxk = psi_idx[2].reshape(K, NLAT, NNZ)
  valsk = psi_vals.reshape(K, NLAT, NNZ)
  iflat = jnp.transpose(idxk, (1, 0, 2)).reshape(NSEG, NNZ)
  vals = jnp.transpose(valsk, (1, 0, 2)).reshape(NSEG, NNZ)
  iflat_x = jnp.broadcast_to(
      iflat[SEG_BASE:, :, None], (SC_SEGS, NNZ, LANES))
  vals_x = jnp.broadcast_to(
      vals[SEG_BASE:, :, None], (SC_SEGS, NNZ, LANES))

  w2 = jnp.transpose(weight, (0, 2, 1)).reshape(COUT, K * CIN)
  wkt = jnp.transpose(weight, (2, 1, 0))
  perm = jnp.array(
      [32 * u + 2 * i + h
       for u in range(2) for h in range(2) for i in range(16)],
      dtype=jnp.int32)
  wkt_sc = wkt[:, perm, :]
  bias2 = bias[:, None]

  mesh = plsc.VectorSubcoreMesh(
      core_axis_name="c", subcore_axis_name="s",
      num_cores=2, num_subcores=16)
  y_sc = pl.kernel(
      _sc_stage_a,
      out_type=jax.ShapeDtypeStruct((SC_SEGS, NLON, CIN), jnp.float32),
      mesh=mesh,
      scratch_types=[
          pltpu.VMEM((NNZ, LANES), jnp.int32),
          pltpu.VMEM((NNZ, LANES), jnp.float32),
          pltpu.VMEM((2, NROWS), jnp.int32),
          pltpu.VMEM((2, NROWS, ROWLEN), jnp.int32),
          pltpu.VMEM((NLON, CIN), jnp.float32),
          pltpu.SemaphoreType.DMA,
          pltpu.SemaphoreType.DMA,
      ],
  )(xp, iflat_x, vals_x)

  out_tc = pl.pallas_call(
      _disco_tc_kernel,
      grid=(B_TC,),
      in_specs=[
          pl.BlockSpec((K, LAT_BLK, NNZ), lambda i: (0, i, 0),
                       memory_space=pltpu.SMEM),
          pl.BlockSpec((K, LAT_BLK, NNZ), lambda i: (0, i, 0),
                       memory_space=pltpu.SMEM),
          pl.BlockSpec((NLAT, CIN, NLON), lambda i: (0, 0, 0)),
          pl.BlockSpec((COUT, K * CIN), lambda i: (0, 0)),
          pl.BlockSpec((COUT, 1), lambda i: (0, 0)),
      ],
      out_specs=pl.BlockSpec((COUT, LAT_BLK * NLON), lambda i: (0, i)),
      out_shape=jax.ShapeDtypeStruct((COUT, B_TC * LAT_BLK * NLON),
                                     jnp.float32),
      scratch_shapes=[pltpu.VMEM((K * CIN, LAT_BLK * NLON), jnp.float32)],
  )(idxk, valsk, xt, w2, bias2)

  out_sc = pl.pallas_call(
      _mix_kernel,
      grid=(NLAT // LAT_BLK - B_TC,),
      in_specs=[
          pl.BlockSpec((K * LAT_BLK, NLON, CIN), lambda i: (i, 0, 0)),
          pl.BlockSpec((K, CIN, COUT), lambda i: (0, 0, 0)),
          pl.BlockSpec((COUT, 1), lambda i: (0, 0)),
      ],
      out_specs=pl.BlockSpec((COUT, LAT_BLK * NLON), lambda i: (0, i)),
      out_shape=jax.ShapeDtypeStruct(
          (COUT, (NLAT - B_TC * LAT_BLK) * NLON), jnp.float32),
  )(y_sc, wkt_sc, bias2)

  out2d = jnp.concatenate([out_tc, out_sc], axis=1)
  return out2d.reshape(1, COUT, NLAT, NLON)


# speedup vs baseline: 1.4365x; 1.4365x over previous
import functools
import jax
import jax.numpy as jnp
from jax import lax
from jax.experimental import pallas as pl
from jax.experimental.pallas import tpu as pltpu
from jax.experimental.pallas import tpu_sc as plsc

K = 3
NLAT = 128
NLON = 256
NNZ = 32
CIN = 64
COUT = 64
NSEG = K * NLAT
NWORKERS = 32
NROWS = NLON // 2
ROWLEN = 2 * CIN
LANES = 16
LAT_BLK = 8
B_TC = 10
SEG_BASE = K * LAT_BLK * B_TC
SC_SEGS = NSEG - SEG_BASE


def _disco_tc_kernel(idx_ref, val_ref, xt_ref, w2_ref, bias_ref, out_ref,
                     y_ref):
  UNROLL = 8
  for k in range(K):
    for j in range(LAT_BLK):
      def body(n2, accs):
        a0, a1 = accs
        for u in range(UNROLL):
          n = n2 * UNROLL + u
          iflat = idx_ref[k, j, n]
          lat = iflat >> 8
          shift = (NLON - (iflat & (NLON - 1))) & (NLON - 1)
          val = val_ref[k, j, n]
          t = val * pltpu.roll(xt_ref[lat], shift, 1)
          if u % 2 == 0:
            a0 = a0 + t
          else:
            a1 = a1 + t
        return (a0, a1)

      z = jnp.zeros((CIN, NLON), jnp.float32)
      a0, a1 = lax.fori_loop(0, NNZ // UNROLL, body, (z, z))
      y_ref[k * CIN:(k + 1) * CIN, j * NLON:(j + 1) * NLON] = a0 + a1
  out_ref[...] = (
      jnp.dot(w2_ref[...], y_ref[...], preferred_element_type=jnp.float32)
      + bias_ref[...]
  )


def _sc_stage_a(xp, iflat_x, vals_x, y_out,
                if_v, val_v, idx_b, gbuf, acc, sem0, sem1):
  wid = lax.axis_index("s") * 2 + lax.axis_index("c")
  iota = lax.iota(jnp.int32, LANES)
  sems = (sem0, sem1)
  n_full = SC_SEGS // NWORKERS
  rem = SC_SEGS % NWORKERS
  n_t = n_full + jnp.where(wid < rem, 1, 0)

  def build_and_fire(e, slot):
    isp = if_v[e]
    lat_hi = (isp >> 8) << 8
    s_s = isp & (NLON - 1)
    for c in range(NROWS // LANES):
      idx_b[slot, pl.ds(c * LANES, LANES)] = (
          lat_hi | ((s_s + 2 * (c * LANES + iota)) & (NLON - 1)))
    pltpu.make_async_copy(
        xp.at[idx_b.at[slot]], gbuf.at[slot], sems[slot]).start()

  def wait(slot):
    pltpu.make_async_copy(
        xp.at[idx_b.at[slot]], gbuf.at[slot], sems[slot]).wait()

  def fma(e, slot):
    vsp = val_v[e]

    @plsc.parallel_loop(0, NROWS, unroll=8)
    def chunk_body(i):
      for par in range(2):
        for c in range(CIN // LANES):
          g = gbuf[slot, i, pl.ds(par * CIN + c * LANES, LANES)]
          plsc.addupdate(
              acc.at[2 * i + par, pl.ds(c * LANES, LANES)], g * vsp)

  def seg_body(t, _):
    seg = t * NWORKERS + wid
    pltpu.sync_copy(iflat_x.at[seg], if_v)
    pltpu.sync_copy(vals_x.at[seg], val_v)

    zeros = jnp.zeros((LANES,), jnp.float32)

    @plsc.parallel_loop(0, NLON, unroll=8)
    def zero_body(r):
      for c in range(CIN // LANES):
        acc[r, pl.ds(c * LANES, LANES)] = zeros

    build_and_fire(0, 0)
    def pair_body(m, _):
      e0 = 2 * m
      e1 = e0 + 1
      build_and_fire(e1, 1)
      wait(0)
      fma(e0, 0)

      @pl.when(e1 < NNZ - 1)
      def _():
        build_and_fire(e1 + 1, 0)
      wait(1)
      fma(e1, 1)
      return 0
    lax.fori_loop(0, NNZ // 2, pair_body, 0)

    pltpu.sync_copy(acc, y_out.at[seg])
    return 0

  lax.fori_loop(0, n_t, seg_body, 0)


def _mix_kernel(y_ref, wkt_ref, bias_ref, out_ref):
  for j in range(LAT_BLK):
    acc = None
    for k in range(K):
      t2 = y_ref[j * K + k]
      p = jnp.dot(t2, wkt_ref[k], preferred_element_type=jnp.float32)
      acc = p if acc is None else acc + p
    out_ref[:, j * NLON:(j + 1) * NLON] = acc.T + bias_ref[...]


@jax.jit
def kernel(x, psi_idx, psi_vals, weight, bias):
  xt = jnp.transpose(x[0], (1, 0, 2))
  xlc = jnp.transpose(x[0], (1, 2, 0))
  xpair = jnp.concatenate(
      [xlc, jnp.roll(xlc, -1, axis=1)], axis=-1)
  xp = xpair.reshape(NLAT * NLON, ROWLEN)

  idxk = psi_idx[2].reshape(K, NLAT, NNZ)
  valsk = psi_vals.reshape(K, NLAT, NNZ)
  iflat = jnp.transpose(idxk, (1, 0, 2)).reshape(NSEG, NNZ)
  vals = jnp.transpose(valsk, (1, 0, 2)).reshape(NSEG, NNZ)
  iflat_x = jnp.broadcast_to(
      iflat[SEG_BASE:, :, None], (SC_SEGS, NNZ, LANES))
  vals_x = jnp.broadcast_to(
      vals[SEG_BASE:, :, None], (SC_SEGS, NNZ, LANES))

  w2 = jnp.transpose(weight, (0, 2, 1)).reshape(COUT, K * CIN)
  wkt = jnp.transpose(weight, (2, 1, 0))
  bias2 = bias[:, None]

  mesh = plsc.VectorSubcoreMesh(
      core_axis_name="c", subcore_axis_name="s",
      num_cores=2, num_subcores=16)
  y_sc = pl.kernel(
      _sc_stage_a,
      out_type=jax.ShapeDtypeStruct((SC_SEGS, NLON, CIN), jnp.float32),
      mesh=mesh,
      scratch_types=[
          pltpu.VMEM((NNZ, LANES), jnp.int32),
          pltpu.VMEM((NNZ, LANES), jnp.float32),
          pltpu.VMEM((2, NROWS), jnp.int32),
          pltpu.VMEM((2, NROWS, ROWLEN), jnp.float32),
          pltpu.VMEM((NLON, CIN), jnp.float32),
          pltpu.SemaphoreType.DMA,
          pltpu.SemaphoreType.DMA,
      ],
  )(xp, iflat_x, vals_x)

  out_tc = pl.pallas_call(
      _disco_tc_kernel,
      grid=(B_TC,),
      in_specs=[
          pl.BlockSpec((K, LAT_BLK, NNZ), lambda i: (0, i, 0),
                       memory_space=pltpu.SMEM),
          pl.BlockSpec((K, LAT_BLK, NNZ), lambda i: (0, i, 0),
                       memory_space=pltpu.SMEM),
          pl.BlockSpec((NLAT, CIN, NLON), lambda i: (0, 0, 0)),
          pl.BlockSpec((COUT, K * CIN), lambda i: (0, 0)),
          pl.BlockSpec((COUT, 1), lambda i: (0, 0)),
      ],
      out_specs=pl.BlockSpec((COUT, LAT_BLK * NLON), lambda i: (0, i)),
      out_shape=jax.ShapeDtypeStruct((COUT, B_TC * LAT_BLK * NLON),
                                     jnp.float32),
      scratch_shapes=[pltpu.VMEM((K * CIN, LAT_BLK * NLON), jnp.float32)],
  )(idxk, valsk, xt, w2, bias2)

  out_sc = pl.pallas_call(
      _mix_kernel,
      grid=(NLAT // LAT_BLK - B_TC,),
      in_specs=[
          pl.BlockSpec((K * LAT_BLK, NLON, CIN), lambda i: (i, 0, 0)),
          pl.BlockSpec((K, CIN, COUT), lambda i: (0, 0, 0)),
          pl.BlockSpec((COUT, 1), lambda i: (0, 0)),
      ],
      out_specs=pl.BlockSpec((COUT, LAT_BLK * NLON), lambda i: (0, i)),
      out_shape=jax.ShapeDtypeStruct(
          (COUT, (NLAT - B_TC * LAT_BLK) * NLON), jnp.float32),
  )(y_sc, wkt, bias2)

  out2d = jnp.concatenate([out_tc, out_sc], axis=1)
  return out2d.reshape(1, COUT, NLAT, NLON)
